# baseline (speedup 1.0000x reference)
import functools

import jax
import jax.numpy as jnp
from jax import lax
from jax.experimental import pallas as pl
from jax.experimental.pallas import tpu as pltpu
from jax.experimental.pallas import tpu_sc as plsc

N = 10000
E = 320000
D_IN = 128
H = 128
D_OUT = 64
B = 2048

NC = 2
NS = 16
NW = NC * NS

EPW = E // NS
CH = 128
NFULL = EPW // CH
ETAIL = EPW - NFULL * CH
K = 2
NG = NFULL // K
RPT = 624
TAIL0 = NS * RPT
TAIL = N - TAIL0
BPW = 2 * B // NW

_MESH = plsc.VectorSubcoreMesh(core_axis_name="c", subcore_axis_name="s")


def _spmm_body(with_deg, h_hbm, src_hbm, dst_hbm, zeros_hbm, zeros1d_hbm,
               agg_out, hist_out, sidx, didx, rows, sidx_t, didx_t, rows_t,
               hist, agg_s, isem, *gssems):
    gsems = gssems[:K]
    ssems = gssems[K:]
    c = lax.axis_index("c")
    s = lax.axis_index("s")

    r0 = pl.multiple_of(s * RPT, 8)
    pltpu.sync_copy(zeros_hbm.at[pl.ds(r0, RPT)], agg_s.at[pl.ds(r0, RPT)])
    if with_deg:
        pltpu.sync_copy(zeros1d_hbm, hist)

    @pl.when(s == NS - 1)
    def _zero_tail():
        pltpu.sync_copy(zeros_hbm.at[pl.ds(TAIL0, TAIL)],
                        agg_s.at[pl.ds(TAIL0, TAIL)])

    plsc.subcore_barrier()

    ebase = pl.multiple_of(c * E + s * EPW, 8)
    ones16 = jnp.ones((16,), jnp.float32)

    def group(gi, carry):
        for k in range(K):
            @pl.when(gi > 0)
            def _drain(k=k):
                pltpu.make_async_copy(rows.at[k], agg_s.at[didx.at[k]],
                                      ssems[k]).wait()
        gbase = pl.multiple_of(ebase + gi * (K * CH), 8)
        idesc = []
        for k in range(K):
            off = pl.multiple_of(gbase + k * CH, 8)
            idesc.append(pltpu.async_copy(src_hbm.at[pl.ds(off, CH)],
                                          sidx.at[k], isem))
            idesc.append(pltpu.async_copy(dst_hbm.at[pl.ds(off, CH)],
                                          didx.at[k], isem))
        for d in idesc:
            d.wait()
        gdesc = [pltpu.async_copy(h_hbm.at[sidx.at[k]], rows.at[k], gsems[k])
                 for k in range(K)]
        if with_deg:
            for k in range(K):
                for j in range(CH // 16):
                    plsc.addupdate_scatter(hist, [didx[k, pl.ds(j * 16, 16)]],
                                           ones16)
        for k in range(K):
            gdesc[k].wait()
            pltpu.async_copy(rows.at[k], agg_s.at[didx.at[k]], ssems[k],
                             add=True)
        return carry

    lax.fori_loop(0, NG, group, 0)

    toff = pl.multiple_of(ebase + NFULL * CH, 8)
    t1 = pltpu.async_copy(src_hbm.at[pl.ds(toff, ETAIL)], sidx_t, isem)
    t2 = pltpu.async_copy(dst_hbm.at[pl.ds(toff, ETAIL)], didx_t, isem)
    t1.wait()
    t2.wait()
    tg = pltpu.async_copy(h_hbm.at[sidx_t], rows_t, isem)
    if with_deg:
        for j in range(ETAIL // 16):
            plsc.addupdate_scatter(hist, [didx_t[pl.ds(j * 16, 16)]], ones16)
    tg.wait()
    pltpu.sync_copy(rows_t, agg_s.at[didx_t], add=True)

    for k in range(K):
        pltpu.make_async_copy(rows.at[k], agg_s.at[didx.at[k]],
                              ssems[k]).wait()
    plsc.subcore_barrier()

    pltpu.sync_copy(agg_s.at[pl.ds(r0, RPT)], agg_out.at[c, pl.ds(r0, RPT)])
    if with_deg:
        pltpu.sync_copy(hist, hist_out.at[pl.ds((c * NS + s) * N, N)])

    @pl.when(s == NS - 1)
    def _write_tail():
        pltpu.sync_copy(agg_s.at[pl.ds(TAIL0, TAIL)],
                        agg_out.at[c, pl.ds(TAIL0, TAIL)])


def _make_spmm(with_deg):
    out_type = [jax.ShapeDtypeStruct((NC, N, H), jnp.float32),
                jax.ShapeDtypeStruct((NW * N,), jnp.float32)]
    scratch = [
        pltpu.VMEM((K, CH), jnp.int32),
        pltpu.VMEM((K, CH), jnp.int32),
        pltpu.VMEM((K, CH, H), jnp.float32),
        pltpu.VMEM((ETAIL,), jnp.int32),
        pltpu.VMEM((ETAIL,), jnp.int32),
        pltpu.VMEM((ETAIL, H), jnp.float32),
        pltpu.VMEM((N,), jnp.float32),
        pltpu.VMEM_SHARED((N, H), jnp.float32),
        pltpu.SemaphoreType.DMA,
    ] + [pltpu.SemaphoreType.DMA] * (2 * K)
    return pl.kernel(functools.partial(_spmm_body, with_deg),
                     out_type=out_type, mesh=_MESH, scratch_types=scratch,
                     compiler_params=pltpu.CompilerParams(
                         needs_layout_passes=False))


_spmm_deg = _make_spmm(True)
_spmm = _make_spmm(False)


def _gather_body(h_hbm, tgt_hbm, out_hbm, tidx, rows, sem):
    c = lax.axis_index("c")
    s = lax.axis_index("s")
    base = pl.multiple_of((c * NS + s) * BPW, 8)
    pltpu.sync_copy(tgt_hbm.at[pl.ds(base, BPW)], tidx)
    pltpu.async_copy(h_hbm.at[tidx], rows, sem).wait()
    pltpu.sync_copy(rows, out_hbm.at[pl.ds(base, BPW)])


_gather_tgt = pl.kernel(
    _gather_body,
    out_type=jax.ShapeDtypeStruct((2 * B, H), jnp.float32),
    mesh=_MESH,
    scratch_types=[
        pltpu.VMEM((BPW,), jnp.int32),
        pltpu.VMEM((BPW, H), jnp.float32),
        pltpu.SemaphoreType.DMA,
    ],
)


def _mlp_bn_body(nf1, nf2, Wm1, Wm2, bm1, bm2, g1, g2, be1, be2, out):
    for p, (nf, Wm, bm, g, be) in enumerate(
            ((nf1, Wm1, bm1, g1, be1), (nf2, Wm2, bm2, g2, be2))):
        h = jnp.dot(nf[...], Wm[...], preferred_element_type=jnp.float32) \
            + bm[...]
        h = jnp.where(h > 0, h, 0.1 * h)
        mu = jnp.mean(h, axis=0, keepdims=True)
        xc = h - mu
        var = jnp.mean(xc * xc, axis=0, keepdims=True)
        out[p] = xc * lax.rsqrt(var + 1e-5) * g[...] + be[...]


_mlp_bn = pl.pallas_call(
    _mlp_bn_body,
    out_shape=jax.ShapeDtypeStruct((NC, N, H), jnp.float32),
)


def _layer_body(h, agg, dcols, Ws1, Wn1, b1, Ws2, Wn2, b2, out):
    for p, (Ws, Wn, b) in enumerate(((Ws1, Wn1, b1), (Ws2, Wn2, b2))):
        deg = jnp.maximum(
            jnp.sum(dcols[p], axis=1, keepdims=True), 1.0)
        a = agg[p] / deg
        out[p] = jnp.maximum(
            jnp.dot(h[p], Ws[...], preferred_element_type=jnp.float32)
            + jnp.dot(a, Wn[...], preferred_element_type=jnp.float32)
            + b[...], 0.0)


_layer = pl.pallas_call(
    _layer_body,
    out_shape=jax.ShapeDtypeStruct((NC, N, H), jnp.float32),
)


def _loss_body(ht, x1, x2, Wp1, Wp2, bp1, bp2, out):
    for p, (x, Wp, bp) in enumerate(((x1, Wp1, bp1), (x2, Wp2, bp2))):
        xp = jnp.dot(ht[pl.ds(p * B, B)], Wp[...],
                     preferred_element_type=jnp.float32) + bp[...]
        r = xp - x[...]
        out[pl.ds(p, 1)] = jnp.sum(r * r, keepdims=True) * (1.0 / (B * D_OUT))


_loss = pl.pallas_call(
    _loss_body,
    out_shape=jax.ShapeDtypeStruct((2, 1), jnp.float32),
)


def kernel(node_feat1, edge_index1, tgt1, x1, Wm1, bm1, g1, be1,
           Wsa1, Wna1, ba1, Wsb1, Wnb1, bb1, Wp1, bp1,
           node_feat2, edge_index2, tgt2, x2, Wm2, bm2, g2, be2,
           Wsa2, Wna2, ba2, Wsb2, Wnb2, bb2, Wp2, bp2):
    src = jnp.concatenate([edge_index1[0], edge_index2[0] + N])
    dst = jnp.concatenate([edge_index1[1], edge_index2[1]])
    tgt = jnp.concatenate([tgt1, tgt2 + N])
    zeros = jnp.zeros((N, H), jnp.float32)
    zeros1d = jnp.zeros((N,), jnp.float32)

    r = lambda v: v.reshape(1, -1)
    h = _mlp_bn(node_feat1, node_feat2, Wm1, Wm2, r(bm1), r(bm2),
                r(g1), r(g2), r(be1), r(be2))
    agg, hists = _spmm_deg(h.reshape(2 * N, H), src, dst, zeros, zeros1d)
    dcols = jnp.transpose(hists.reshape(NC, NS, N), (0, 2, 1))
    h = _layer(h, agg, dcols, Wsa1, Wna1, r(ba1), Wsa2, Wna2, r(ba2))
    agg, _ = _spmm(h.reshape(2 * N, H), src, dst, zeros, zeros1d)
    h = _layer(h, agg, dcols, Wsb1, Wnb1, r(bb1), Wsb2, Wnb2, r(bb2))
    ht = _gather_tgt(h.reshape(2 * N, H), tgt)
    out = _loss(ht, x1, x2, Wp1, Wp2, r(bp1), r(bp2))
    return out.reshape(2)


# speedup vs baseline: 1.1221x; 1.1221x over previous
import functools

import jax
import jax.numpy as jnp
from jax import lax
from jax.experimental import pallas as pl
from jax.experimental.pallas import tpu as pltpu
from jax.experimental.pallas import tpu_sc as plsc

N = 10000
E = 320000
D_IN = 128
H = 128
D_OUT = 64
B = 2048

NC = 2
NS = 16
NW = NC * NS

EPW = E // NS
CH = 128
NFULL = EPW // CH
ETAIL = EPW - NFULL * CH
K = 2
NG = NFULL // K
RPT = 624
TAIL0 = NS * RPT
TAIL = N - TAIL0
BPW = 2 * B // NW

_MESH = plsc.VectorSubcoreMesh(core_axis_name="c", subcore_axis_name="s")


def _spmm_body(with_deg, h_hbm, src_hbm, dst_hbm, zeros_hbm, zeros1d_hbm,
               agg_out, hist_out, sidx, didx, rows, sidx_t, didx_t, rows_t,
               hist, agg_s, isem, *gssems):
    gsems = gssems[:K]
    ssems = gssems[K:]
    c = lax.axis_index("c")
    s = lax.axis_index("s")

    r0 = pl.multiple_of(s * RPT, 8)
    pltpu.sync_copy(zeros_hbm.at[pl.ds(r0, RPT)], agg_s.at[pl.ds(r0, RPT)])
    if with_deg:
        pltpu.sync_copy(zeros1d_hbm, hist)

    @pl.when(s == NS - 1)
    def _zero_tail():
        pltpu.sync_copy(zeros_hbm.at[pl.ds(TAIL0, TAIL)],
                        agg_s.at[pl.ds(TAIL0, TAIL)])

    plsc.subcore_barrier()

    ebase = pl.multiple_of(c * E + s * EPW, 8)
    ones16 = jnp.ones((16,), jnp.float32)

    def issue_idx(gi, par):
        gbase = pl.multiple_of(ebase + gi * (K * CH), 8)
        for k in range(K):
            off = pl.multiple_of(gbase + k * CH, 8)
            pltpu.async_copy(src_hbm.at[pl.ds(off, CH)], sidx.at[par, k], isem)
            pltpu.async_copy(dst_hbm.at[pl.ds(off, CH)], didx.at[par, k], isem)

    def one_group(gi, par):
        for k in range(K):
            pltpu.make_async_copy(src_hbm.at[pl.ds(0, CH)],
                                  sidx.at[par, k], isem).wait()
            pltpu.make_async_copy(dst_hbm.at[pl.ds(0, CH)],
                                  didx.at[par, k], isem).wait()

        @pl.when(gi < NG - 1)
        def _prefetch():
            issue_idx(gi + 1, 1 - par)

        for k in range(K):
            @pl.when(gi > 0)
            def _drain(k=k):
                pltpu.make_async_copy(rows.at[k], agg_s.at[didx.at[par, k]],
                                      ssems[k]).wait()
        gdesc = [pltpu.async_copy(h_hbm.at[sidx.at[par, k]], rows.at[k],
                                  gsems[k]) for k in range(K)]
        if with_deg:
            for k in range(K):
                for j in range(CH // 16):
                    plsc.addupdate_scatter(
                        hist, [didx[par, k, pl.ds(j * 16, 16)]], ones16)
        for k in range(K):
            gdesc[k].wait()
            pltpu.async_copy(rows.at[k], agg_s.at[didx.at[par, k]], ssems[k],
                             add=True)

    issue_idx(0, 0)

    def dgroup(di, carry):
        one_group(di * 2, 0)
        one_group(di * 2 + 1, 1)
        return carry

    lax.fori_loop(0, NG // 2, dgroup, 0)

    toff = pl.multiple_of(ebase + NFULL * CH, 8)
    t1 = pltpu.async_copy(src_hbm.at[pl.ds(toff, ETAIL)], sidx_t, isem)
    t2 = pltpu.async_copy(dst_hbm.at[pl.ds(toff, ETAIL)], didx_t, isem)
    t1.wait()
    t2.wait()
    tg = pltpu.async_copy(h_hbm.at[sidx_t], rows_t, isem)
    if with_deg:
        for j in range(ETAIL // 16):
            plsc.addupdate_scatter(hist, [didx_t[pl.ds(j * 16, 16)]], ones16)
    tg.wait()
    pltpu.sync_copy(rows_t, agg_s.at[didx_t], add=True)

    for k in range(K):
        pltpu.make_async_copy(rows.at[k], agg_s.at[didx.at[1, k]],
                              ssems[k]).wait()
    plsc.subcore_barrier()

    pltpu.sync_copy(agg_s.at[pl.ds(r0, RPT)], agg_out.at[c, pl.ds(r0, RPT)])
    if with_deg:
        pltpu.sync_copy(hist, hist_out.at[pl.ds((c * NS + s) * N, N)])

    @pl.when(s == NS - 1)
    def _write_tail():
        pltpu.sync_copy(agg_s.at[pl.ds(TAIL0, TAIL)],
                        agg_out.at[c, pl.ds(TAIL0, TAIL)])


def _make_spmm(with_deg):
    out_type = [jax.ShapeDtypeStruct((NC, N, H), jnp.float32),
                jax.ShapeDtypeStruct((NW * N,), jnp.float32)]
    scratch = [
        pltpu.VMEM((2, K, CH), jnp.int32),
        pltpu.VMEM((2, K, CH), jnp.int32),
        pltpu.VMEM((K, CH, H), jnp.float32),
        pltpu.VMEM((ETAIL,), jnp.int32),
        pltpu.VMEM((ETAIL,), jnp.int32),
        pltpu.VMEM((ETAIL, H), jnp.float32),
        pltpu.VMEM((N,), jnp.float32),
        pltpu.VMEM_SHARED((N, H), jnp.float32),
        pltpu.SemaphoreType.DMA,
    ] + [pltpu.SemaphoreType.DMA] * (2 * K)
    return pl.kernel(functools.partial(_spmm_body, with_deg),
                     out_type=out_type, mesh=_MESH, scratch_types=scratch,
                     compiler_params=pltpu.CompilerParams(
                         needs_layout_passes=False))


_spmm_deg = _make_spmm(True)
_spmm = _make_spmm(False)


def _gather_body(h_hbm, tgt_hbm, out_hbm, tidx, rows, sem):
    c = lax.axis_index("c")
    s = lax.axis_index("s")
    base = pl.multiple_of((c * NS + s) * BPW, 8)
    pltpu.sync_copy(tgt_hbm.at[pl.ds(base, BPW)], tidx)
    pltpu.async_copy(h_hbm.at[tidx], rows, sem).wait()
    pltpu.sync_copy(rows, out_hbm.at[pl.ds(base, BPW)])


_gather_tgt = pl.kernel(
    _gather_body,
    out_type=jax.ShapeDtypeStruct((2 * B, H), jnp.float32),
    mesh=_MESH,
    scratch_types=[
        pltpu.VMEM((BPW,), jnp.int32),
        pltpu.VMEM((BPW, H), jnp.float32),
        pltpu.SemaphoreType.DMA,
    ],
)


def _mlp_bn_body(nf1, nf2, Wm1, Wm2, bm1, bm2, g1, g2, be1, be2, out):
    for p, (nf, Wm, bm, g, be) in enumerate(
            ((nf1, Wm1, bm1, g1, be1), (nf2, Wm2, bm2, g2, be2))):
        h = jnp.dot(nf[...], Wm[...], preferred_element_type=jnp.float32) \
            + bm[...]
        h = jnp.where(h > 0, h, 0.1 * h)
        mu = jnp.mean(h, axis=0, keepdims=True)
        xc = h - mu
        var = jnp.mean(xc * xc, axis=0, keepdims=True)
        out[p] = xc * lax.rsqrt(var + 1e-5) * g[...] + be[...]


_mlp_bn = pl.pallas_call(
    _mlp_bn_body,
    out_shape=jax.ShapeDtypeStruct((NC, N, H), jnp.float32),
)


def _layer_body(h, agg, dcols, Ws1, Wn1, b1, Ws2, Wn2, b2, out):
    for p, (Ws, Wn, b) in enumerate(((Ws1, Wn1, b1), (Ws2, Wn2, b2))):
        deg = jnp.maximum(
            jnp.sum(dcols[p], axis=1, keepdims=True), 1.0)
        a = agg[p] / deg
        out[p] = jnp.maximum(
            jnp.dot(h[p], Ws[...], preferred_element_type=jnp.float32)
            + jnp.dot(a, Wn[...], preferred_element_type=jnp.float32)
            + b[...], 0.0)


_layer = pl.pallas_call(
    _layer_body,
    out_shape=jax.ShapeDtypeStruct((NC, N, H), jnp.float32),
)


def _loss_body(ht, x1, x2, Wp1, Wp2, bp1, bp2, out):
    for p, (x, Wp, bp) in enumerate(((x1, Wp1, bp1), (x2, Wp2, bp2))):
        xp = jnp.dot(ht[pl.ds(p * B, B)], Wp[...],
                     preferred_element_type=jnp.float32) + bp[...]
        r = xp - x[...]
        out[pl.ds(p, 1)] = jnp.sum(r * r, keepdims=True) * (1.0 / (B * D_OUT))


_loss = pl.pallas_call(
    _loss_body,
    out_shape=jax.ShapeDtypeStruct((2, 1), jnp.float32),
)


def kernel(node_feat1, edge_index1, tgt1, x1, Wm1, bm1, g1, be1,
           Wsa1, Wna1, ba1, Wsb1, Wnb1, bb1, Wp1, bp1,
           node_feat2, edge_index2, tgt2, x2, Wm2, bm2, g2, be2,
           Wsa2, Wna2, ba2, Wsb2, Wnb2, bb2, Wp2, bp2):
    src = jnp.concatenate([edge_index1[0], edge_index2[0] + N])
    dst = jnp.concatenate([edge_index1[1], edge_index2[1]])
    tgt = jnp.concatenate([tgt1, tgt2 + N])
    zeros = jnp.zeros((N, H), jnp.float32)
    zeros1d = jnp.zeros((N,), jnp.float32)

    r = lambda v: v.reshape(1, -1)
    h = _mlp_bn(node_feat1, node_feat2, Wm1, Wm2, r(bm1), r(bm2),
                r(g1), r(g2), r(be1), r(be2))
    agg, hists = _spmm_deg(h.reshape(2 * N, H), src, dst, zeros, zeros1d)
    dcols = jnp.transpose(hists.reshape(NC, NS, N), (0, 2, 1))
    h = _layer(h, agg, dcols, Wsa1, Wna1, r(ba1), Wsa2, Wna2, r(ba2))
    agg, _ = _spmm(h.reshape(2 * N, H), src, dst, zeros, zeros1d)
    h = _layer(h, agg, dcols, Wsb1, Wnb1, r(bb1), Wsb2, Wnb2, r(bb2))
    ht = _gather_tgt(h.reshape(2 * N, H), tgt)
    out = _loss(ht, x1, x2, Wp1, Wp2, r(bp1), r(bp2))
    return out.reshape(2)
